# baseline (speedup 1.0000x reference)
import functools

import jax
import jax.numpy as jnp
from jax import lax
from jax.experimental import pallas as pl
from jax.experimental.pallas import tpu as pltpu
from jax.experimental.pallas import tpu_sc as plsc

B, M, N = 64, 32, 8192
K = 16


def _select_kernel(g_ref, s_ref, eg_ref, vals_ref, idx_ref, sel_ref, s0p_ref):
    x = g_ref[...]
    iota = lax.broadcasted_iota(jnp.int32, (B, N), 1)
    vals = []
    idxs = []
    for _ in range(K):
        m = jnp.max(x, axis=1, keepdims=True)
        idx = jnp.min(jnp.where(x >= m, iota, N), axis=1, keepdims=True)
        vals.append(m)
        idxs.append(idx)
        x = jnp.where(iota == idx, -1.0, x)
    vals_ref[...] = jnp.maximum(jnp.concatenate(vals, axis=1), 1e-8)
    idx_ref[...] = jnp.concatenate(idxs, axis=1)

    sp = s_ref[...]
    sc = sp * eg_ref[...]
    sm = jnp.max(sc, axis=1, keepdims=True)
    iota2 = lax.broadcasted_iota(jnp.int32, (K, N), 1)
    sel = jnp.min(jnp.where(sc >= sm, iota2, N), axis=1)
    sel_ref[0, :] = sel
    s0p_ref[0, :] = jnp.sum(jnp.where(iota2 == sel[:, None], sp, 0.0), axis=1)


def _gather_kernel(sel_ref, p_ref, out_ref):
    i = pl.program_id(0)
    row = i % 8
    col = sel_ref[i] % 128
    blk = p_ref[...]
    rmask = lax.broadcasted_iota(jnp.int32, (B, 8, 128), 1) == row
    cmask = lax.broadcasted_iota(jnp.int32, (B, 8, 128), 2) == col
    v = jnp.sum(jnp.where(rmask & cmask, blk, 0.0), axis=(1, 2))
    out_ref[0, 0, :] = jnp.maximum(v, 1e-8)


@jax.jit
def kernel(probs):
    eg = jnp.exp(jax.random.gumbel(jax.random.key(42), (K, N), jnp.float32))
    g2 = probs[:, 0, :]
    s2 = probs[0, K:, :]

    vals, idx, sel2d, s0p = pl.pallas_call(
        _select_kernel,
        grid=(),
        in_specs=[
            pl.BlockSpec((B, N), lambda: (0, 0)),
            pl.BlockSpec((K, N), lambda: (0, 0)),
            pl.BlockSpec((K, N), lambda: (0, 0)),
        ],
        out_specs=[
            pl.BlockSpec((B, K), lambda: (0, 0)),
            pl.BlockSpec((B, K), lambda: (0, 0)),
            pl.BlockSpec((1, K), lambda: (0, 0)),
            pl.BlockSpec((1, K), lambda: (0, 0)),
        ],
        out_shape=[
            jax.ShapeDtypeStruct((B, K), jnp.float32),
            jax.ShapeDtypeStruct((B, K), jnp.int32),
            jax.ShapeDtypeStruct((1, K), jnp.int32),
            jax.ShapeDtypeStruct((1, K), jnp.float32),
        ],
    )(g2, s2, eg)

    sel = sel2d[0]

    grid_spec = pltpu.PrefetchScalarGridSpec(
        num_scalar_prefetch=1,
        grid=(K,),
        in_specs=[
            pl.BlockSpec(
                (B, 8, 128),
                lambda i, sr: (0, 2 + i // 8, sr[i] // 128)),
        ],
        out_specs=pl.BlockSpec((1, 1, B), lambda i, sr: (i, 0, 0)),
    )
    sprobs = pl.pallas_call(
        _gather_kernel,
        grid_spec=grid_spec,
        out_shape=jax.ShapeDtypeStruct((K, 1, B), jnp.float32),
    )(sel, probs)

    def _sc_noop(sel_hbm, out_hbm, sel_v, vals_v):
        wid = lax.axis_index("s") * 2 + lax.axis_index("c")
        pltpu.sync_copy(sel_hbm, sel_v)
        for t in range(2):
            b = wid * 2 + t
            vals_v[...] = jnp.maximum(sel_v[...].astype(jnp.float32), 1e-8)
            pltpu.sync_copy(vals_v, out_hbm.at[b])

    sc_probe = functools.partial(
        pl.kernel,
        mesh=plsc.VectorSubcoreMesh(core_axis_name="c", subcore_axis_name="s"),
        compiler_params=pltpu.CompilerParams(needs_layout_passes=False),
        out_type=jax.ShapeDtypeStruct((B, K), jnp.float32),
        scratch_types=[
            pltpu.VMEM((K,), jnp.int32),
            pltpu.VMEM((K,), jnp.float32),
        ],
    )(_sc_noop)
    _ = sc_probe(jnp.arange(K, dtype=jnp.int32))
    sprobs2 = sprobs[:, 0, :].T + 0.0 * _

    selected = jnp.concatenate(
        [idx, jnp.broadcast_to(sel[None, :], (B, K))], axis=1)
    prob = jnp.concatenate([vals, sprobs2], axis=1)
    return selected, prob


# speedup vs baseline: 1.5729x; 1.5729x over previous
import functools

import jax
import jax.numpy as jnp
from jax import lax
from jax.experimental import pallas as pl
from jax.experimental.pallas import tpu as pltpu

B, M, N = 64, 32, 8192
K = 16


def _select_kernel(g_ref, s_ref, eg_ref, vals_ref, idx_ref, sel_ref, s0p_ref):
    x = g_ref[...]
    iota = lax.broadcasted_iota(jnp.int32, (B, N), 1)
    vals = []
    idxs = []
    for _ in range(K):
        m = jnp.max(x, axis=1, keepdims=True)
        idx = jnp.min(jnp.where(x >= m, iota, N), axis=1, keepdims=True)
        vals.append(m)
        idxs.append(idx)
        x = jnp.where(iota == idx, -1.0, x)
    vals_ref[...] = jnp.maximum(jnp.concatenate(vals, axis=1), 1e-8)
    idx_ref[...] = jnp.concatenate(idxs, axis=1)

    sp = s_ref[...]
    sc = sp * eg_ref[...]
    sm = jnp.max(sc, axis=1, keepdims=True)
    iota2 = lax.broadcasted_iota(jnp.int32, (K, N), 1)
    sel = jnp.min(jnp.where(sc >= sm, iota2, N), axis=1)
    sel_ref[0, :] = sel
    s0p_ref[0, :] = jnp.sum(jnp.where(iota2 == sel[:, None], sp, 0.0), axis=1)


def _gather_kernel(sel_ref, p0_ref, p1_ref, p2_ref, p3_ref, out_ref):
    g = pl.program_id(0)
    for j, p_ref in enumerate((p0_ref, p1_ref, p2_ref, p3_ref)):
        i = g * 4 + j
        row = i % 8
        col = sel_ref[i] % 128
        blk = p_ref[...]
        rmask = lax.broadcasted_iota(jnp.int32, (B, 8, 128), 1) == row
        cmask = lax.broadcasted_iota(jnp.int32, (B, 8, 128), 2) == col
        v = jnp.sum(jnp.where(rmask & cmask, blk, 0.0), axis=(1, 2))
        out_ref[j, 0, :] = jnp.maximum(v, 1e-8)


@jax.jit
def kernel(probs):
    eg = jnp.exp(jax.random.gumbel(jax.random.key(42), (K, N), jnp.float32))
    g2 = probs[:, 0, :]
    s2 = probs[0, K:, :]

    vals, idx, sel2d, s0p = pl.pallas_call(
        _select_kernel,
        grid=(),
        in_specs=[
            pl.BlockSpec((B, N), lambda: (0, 0)),
            pl.BlockSpec((K, N), lambda: (0, 0)),
            pl.BlockSpec((K, N), lambda: (0, 0)),
        ],
        out_specs=[
            pl.BlockSpec((B, K), lambda: (0, 0)),
            pl.BlockSpec((B, K), lambda: (0, 0)),
            pl.BlockSpec((1, K), lambda: (0, 0)),
            pl.BlockSpec((1, K), lambda: (0, 0)),
        ],
        out_shape=[
            jax.ShapeDtypeStruct((B, K), jnp.float32),
            jax.ShapeDtypeStruct((B, K), jnp.int32),
            jax.ShapeDtypeStruct((1, K), jnp.int32),
            jax.ShapeDtypeStruct((1, K), jnp.float32),
        ],
    )(g2, s2, eg)

    sel = sel2d[0]

    grid_spec = pltpu.PrefetchScalarGridSpec(
        num_scalar_prefetch=1,
        grid=(4,),
        in_specs=[
            pl.BlockSpec(
                (B, 8, 128),
                lambda g, sr, j=j: (0, 2 + (4 * g + j) // 8,
                                    sr[4 * g + j] // 128))
            for j in range(4)
        ],
        out_specs=pl.BlockSpec((4, 1, B), lambda g, sr: (g, 0, 0)),
    )
    sprobs = pl.pallas_call(
        _gather_kernel,
        grid_spec=grid_spec,
        out_shape=jax.ShapeDtypeStruct((K, 1, B), jnp.float32),
    )(sel, probs, probs, probs, probs)

    selected = jnp.concatenate(
        [idx, jnp.broadcast_to(sel[None, :], (B, K))], axis=1)
    prob = jnp.concatenate([vals, sprobs[:, 0, :].T], axis=1)
    return selected, prob


# speedup vs baseline: 1.6605x; 1.0557x over previous
import functools

import jax
import jax.numpy as jnp
from jax import lax
from jax.experimental import pallas as pl
from jax.experimental.pallas import tpu as pltpu

B, M, N = 64, 32, 8192
K = 16


def _select_kernel(g_ref, s_ref, eg_ref, vals_ref, idx_ref, sel_ref, s0p_ref):
    x = g_ref[...]
    iota = lax.broadcasted_iota(jnp.int32, (B, N), 1)
    vals = []
    idxs = []
    for _ in range(K):
        m = jnp.max(x, axis=1, keepdims=True)
        idx = jnp.min(jnp.where(x >= m, iota, N), axis=1, keepdims=True)
        vals.append(m)
        idxs.append(idx)
        x = jnp.where(iota == idx, -1.0, x)
    vals_ref[...] = jnp.maximum(jnp.concatenate(vals, axis=1), 1e-8)
    idx_ref[...] = jnp.concatenate(idxs, axis=1)

    sp = s_ref[0]
    sc = sp * eg_ref[...]
    sm = jnp.max(sc, axis=1, keepdims=True)
    iota2 = lax.broadcasted_iota(jnp.int32, (K, N), 1)
    sel = jnp.min(jnp.where(sc >= sm, iota2, N), axis=1)
    sel_ref[0, :] = sel
    s0p_ref[0, :] = jnp.sum(jnp.where(iota2 == sel[:, None], sp, 0.0), axis=1)


def _gather_kernel(sel_ref, p0_ref, p1_ref, p2_ref, p3_ref, out_ref):
    g = pl.program_id(0)
    for j, p_ref in enumerate((p0_ref, p1_ref, p2_ref, p3_ref)):
        i = g * 4 + j
        row = i % 8
        col = sel_ref[i] % 128
        blk = p_ref[...]
        rmask = lax.broadcasted_iota(jnp.int32, (B, 8, 128), 1) == row
        cmask = lax.broadcasted_iota(jnp.int32, (B, 8, 128), 2) == col
        v = jnp.sum(jnp.where(rmask & cmask, blk, 0.0), axis=(1, 2))
        out_ref[j, 0, :] = jnp.maximum(v, 1e-8)


@jax.jit
def kernel(probs):
    eg = jnp.exp(jax.random.gumbel(jax.random.key(42), (K, N), jnp.float32))
    g2 = probs[:, 0, :]

    vals, idx, sel2d, s0p = pl.pallas_call(
        _select_kernel,
        grid=(1,),
        in_specs=[
            pl.BlockSpec((B, N), lambda i: (0, 0)),
            pl.BlockSpec((1, K, N), lambda i: (0, 1, 0)),
            pl.BlockSpec((K, N), lambda i: (0, 0)),
        ],
        out_specs=[
            pl.BlockSpec((B, K), lambda i: (0, 0)),
            pl.BlockSpec((B, K), lambda i: (0, 0)),
            pl.BlockSpec((1, K), lambda i: (0, 0)),
            pl.BlockSpec((1, K), lambda i: (0, 0)),
        ],
        out_shape=[
            jax.ShapeDtypeStruct((B, K), jnp.float32),
            jax.ShapeDtypeStruct((B, K), jnp.int32),
            jax.ShapeDtypeStruct((1, K), jnp.int32),
            jax.ShapeDtypeStruct((1, K), jnp.float32),
        ],
    )(g2, probs, eg)

    sel = sel2d[0]

    grid_spec = pltpu.PrefetchScalarGridSpec(
        num_scalar_prefetch=1,
        grid=(4,),
        in_specs=[
            pl.BlockSpec(
                (B, 8, 128),
                lambda g, sr, j=j: (0, 2 + (4 * g + j) // 8,
                                    sr[4 * g + j] // 128))
            for j in range(4)
        ],
        out_specs=pl.BlockSpec((4, 1, B), lambda g, sr: (g, 0, 0)),
    )
    sprobs = pl.pallas_call(
        _gather_kernel,
        grid_spec=grid_spec,
        out_shape=jax.ShapeDtypeStruct((K, 1, B), jnp.float32),
    )(sel, probs, probs, probs, probs)

    selected = jnp.concatenate(
        [idx, jnp.broadcast_to(sel[None, :], (B, K))], axis=1)
    prob = jnp.concatenate([vals, sprobs[:, 0, :].T], axis=1)
    return selected, prob


# speedup vs baseline: 2.2196x; 1.3367x over previous
import functools

import jax
import jax.numpy as jnp
from jax import lax
from jax.experimental import pallas as pl
from jax.experimental.pallas import tpu as pltpu

B, M, N = 64, 32, 8192
K = 16


def _select_kernel(p_ref, s_ref, eg_ref, vals_ref, idx_ref, sel_ref, s0p_ref,
                   g_vmem, dma_sem):
    copy = pltpu.make_async_copy(p_ref.at[:, 0, :], g_vmem, dma_sem)
    copy.start()

    sp = s_ref[0]
    sc = sp * eg_ref[...]
    sm = jnp.max(sc, axis=1, keepdims=True)
    iota2 = lax.broadcasted_iota(jnp.int32, (K, N), 1)
    sel = jnp.min(jnp.where(sc >= sm, iota2, N), axis=1)
    sel_ref[0, :] = sel
    s0p_ref[0, :] = jnp.sum(jnp.where(iota2 == sel[:, None], sp, 0.0), axis=1)

    copy.wait()
    x = g_vmem[...]
    iota = lax.broadcasted_iota(jnp.int32, (B, N), 1)
    vals = []
    idxs = []
    for _ in range(K):
        m = jnp.max(x, axis=1, keepdims=True)
        idx = jnp.min(jnp.where(x >= m, iota, N), axis=1, keepdims=True)
        vals.append(m)
        idxs.append(idx)
        x = jnp.where(iota == idx, -1.0, x)
    vals_ref[...] = jnp.maximum(jnp.concatenate(vals, axis=1), 1e-8)
    idx_ref[...] = jnp.concatenate(idxs, axis=1)


def _gather_kernel(sel_ref, p0_ref, p1_ref, p2_ref, p3_ref, out_ref):
    g = pl.program_id(0)
    for j, p_ref in enumerate((p0_ref, p1_ref, p2_ref, p3_ref)):
        i = g * 4 + j
        row = i % 8
        col = sel_ref[i] % 128
        blk = p_ref[...]
        rmask = lax.broadcasted_iota(jnp.int32, (B, 8, 128), 1) == row
        cmask = lax.broadcasted_iota(jnp.int32, (B, 8, 128), 2) == col
        v = jnp.sum(jnp.where(rmask & cmask, blk, 0.0), axis=(1, 2))
        out_ref[j, 0, :] = jnp.maximum(v, 1e-8)


@jax.jit
def kernel(probs):
    eg = jnp.exp(jax.random.gumbel(jax.random.key(42), (K, N), jnp.float32))

    vals, idx, sel2d, s0p = pl.pallas_call(
        _select_kernel,
        grid=(1,),
        in_specs=[
            pl.BlockSpec(memory_space=pl.ANY),
            pl.BlockSpec((1, K, N), lambda i: (0, 1, 0)),
            pl.BlockSpec((K, N), lambda i: (0, 0)),
        ],
        out_specs=[
            pl.BlockSpec((B, K), lambda i: (0, 0)),
            pl.BlockSpec((B, K), lambda i: (0, 0)),
            pl.BlockSpec((1, K), lambda i: (0, 0)),
            pl.BlockSpec((1, K), lambda i: (0, 0)),
        ],
        out_shape=[
            jax.ShapeDtypeStruct((B, K), jnp.float32),
            jax.ShapeDtypeStruct((B, K), jnp.int32),
            jax.ShapeDtypeStruct((1, K), jnp.int32),
            jax.ShapeDtypeStruct((1, K), jnp.float32),
        ],
        scratch_shapes=[
            pltpu.VMEM((B, N), jnp.float32),
            pltpu.SemaphoreType.DMA,
        ],
    )(probs, probs, eg)

    sel = sel2d[0]

    grid_spec = pltpu.PrefetchScalarGridSpec(
        num_scalar_prefetch=1,
        grid=(4,),
        in_specs=[
            pl.BlockSpec(
                (B, 8, 128),
                lambda g, sr, j=j: (0, 2 + (4 * g + j) // 8,
                                    sr[4 * g + j] // 128))
            for j in range(4)
        ],
        out_specs=pl.BlockSpec((4, 1, B), lambda g, sr: (g, 0, 0)),
    )
    sprobs = pl.pallas_call(
        _gather_kernel,
        grid_spec=grid_spec,
        out_shape=jax.ShapeDtypeStruct((K, 1, B), jnp.float32),
    )(sel, probs, probs, probs, probs)

    selected = jnp.concatenate(
        [idx, jnp.broadcast_to(sel[None, :], (B, K))], axis=1)
    prob = jnp.concatenate([vals, sprobs[:, 0, :].T], axis=1)
    return selected, prob


# speedup vs baseline: 2.4747x; 1.1149x over previous
import functools

import jax
import jax.numpy as jnp
from jax import lax
from jax.experimental import pallas as pl
from jax.experimental.pallas import tpu as pltpu

B, M, N = 64, 32, 8192
K = 16


def _select_kernel(p_ref, s_ref, eg_ref, vals_ref, idx_ref, sel_ref, s0p_ref,
                   g_vmem, dma_sem):
    copy = pltpu.make_async_copy(p_ref.at[:, 0, :], g_vmem, dma_sem)
    copy.start()

    sp = s_ref[0]
    sc = sp * eg_ref[...]
    sm = jnp.max(sc, axis=1, keepdims=True)
    iota2 = lax.broadcasted_iota(jnp.int32, (K, N), 1)
    sel = jnp.min(jnp.where(sc >= sm, iota2, N), axis=1)
    sel_ref[0, :] = sel
    s0p_ref[0, :] = jnp.sum(jnp.where(iota2 == sel[:, None], sp, 0.0), axis=1)

    copy.wait()
    x = g_vmem[...]
    iota = lax.broadcasted_iota(jnp.int32, (B, N), 1)
    vals = []
    idxs = []
    for _ in range(K):
        m = jnp.max(x, axis=1, keepdims=True)
        idx = jnp.min(jnp.where(x >= m, iota, N), axis=1, keepdims=True)
        vals.append(m)
        idxs.append(idx)
        x = jnp.where(iota == idx, -1.0, x)
    vals_ref[...] = jnp.maximum(jnp.concatenate(vals, axis=1), 1e-8)
    idx_ref[...] = jnp.concatenate(idxs, axis=1)


def _gather_kernel(sel_ref, p_ref, out_ref, g_vmem, dma_sem):
    copies = []
    for i in range(K):
        base = (sel_ref[i] // 128) * 128
        c = pltpu.make_async_copy(
            p_ref.at[:, K + i, pl.ds(base, 128)], g_vmem.at[i], dma_sem)
        c.start()
        copies.append(c)
    for c in copies:
        c.wait()
    lane = lax.broadcasted_iota(jnp.int32, (B, 128), 1)
    for i in range(K):
        r = sel_ref[i] % 128
        v = jnp.sum(jnp.where(lane == r, g_vmem[i], 0.0), axis=1)
        out_ref[i, :] = jnp.maximum(v, 1e-8)


@jax.jit
def kernel(probs):
    eg = jnp.exp(jax.random.gumbel(jax.random.key(42), (K, N), jnp.float32))

    vals, idx, sel2d, s0p = pl.pallas_call(
        _select_kernel,
        grid=(1,),
        in_specs=[
            pl.BlockSpec(memory_space=pl.ANY),
            pl.BlockSpec((1, K, N), lambda i: (0, 1, 0)),
            pl.BlockSpec((K, N), lambda i: (0, 0)),
        ],
        out_specs=[
            pl.BlockSpec((B, K), lambda i: (0, 0)),
            pl.BlockSpec((B, K), lambda i: (0, 0)),
            pl.BlockSpec((1, K), lambda i: (0, 0)),
            pl.BlockSpec((1, K), lambda i: (0, 0)),
        ],
        out_shape=[
            jax.ShapeDtypeStruct((B, K), jnp.float32),
            jax.ShapeDtypeStruct((B, K), jnp.int32),
            jax.ShapeDtypeStruct((1, K), jnp.int32),
            jax.ShapeDtypeStruct((1, K), jnp.float32),
        ],
        scratch_shapes=[
            pltpu.VMEM((B, N), jnp.float32),
            pltpu.SemaphoreType.DMA,
        ],
    )(probs, probs, eg)

    sel = sel2d[0]

    grid_spec = pltpu.PrefetchScalarGridSpec(
        num_scalar_prefetch=1,
        grid=(1,),
        in_specs=[pl.BlockSpec(memory_space=pl.ANY)],
        out_specs=pl.BlockSpec((K, B), lambda g, sr: (0, 0)),
        scratch_shapes=[
            pltpu.VMEM((K, B, 128), jnp.float32),
            pltpu.SemaphoreType.DMA,
        ],
    )
    sprobs = pl.pallas_call(
        _gather_kernel,
        grid_spec=grid_spec,
        out_shape=jax.ShapeDtypeStruct((K, B), jnp.float32),
    )(sel, probs)

    selected = jnp.concatenate(
        [idx, jnp.broadcast_to(sel[None, :], (B, K))], axis=1)
    prob = jnp.concatenate([vals, sprobs.T], axis=1)
    return selected, prob
